# Initial kernel scaffold; baseline (speedup 1.0000x reference)
import jax
import jax.numpy as jnp
from jax.experimental import pallas as pl


def kernel(x, edge_index, edge_attr, W_edge_0, b_edge_0, eps_0, W1_0, b1_0, bn1_g_0, bn1_b_0, W2_0, b2_0, bn_g_0, bn_b_0, ln_g_0, ln_b_0, W_edge_1, b_edge_1, eps_1, W1_1, b1_1, bn1_g_1, bn1_b_1, W2_1, b2_1, bn_g_1, bn_b_1, ln_g_1, ln_b_1, W_edge_2, b_edge_2, eps_2, W1_2, b1_2, bn1_g_2, bn1_b_2, W2_2, b2_2, bn_g_2, bn_b_2, ln_g_2, ln_b_2):
    raise NotImplementedError("write your pallas kernel here")


# speedup vs baseline: 2.8742x; 2.8742x over previous
import functools

import jax
import jax.numpy as jnp
from jax import lax
from jax.experimental import pallas as pl
from jax.experimental.pallas import tpu as pltpu
from jax.experimental.pallas import tpu_sc as plsc

N = 10000
E = 320000
D = 128
DE = 16
H = 128

NC = 2
NS = 16
CHUNK = 128
NCHUNKS = E // CHUNK
NTILES = NC * NS
CHUNKS_PER_TILE = -(-NCHUNKS // NTILES)
N_PAD = 10112
ROWS_PER_TILE = N_PAD // NS


def _sc_message_pass_body(h_hbm, e_hbm, src_hbm, dst_hbm, zeros_hbm, out_hbm,
                          src_v, dst_v, rows_v, e_v, agg_sh, sem):
    c = lax.axis_index("c")
    s = lax.axis_index("s")
    wid = c * NS + s
    base = s * ROWS_PER_TILE

    pltpu.sync_copy(zeros_hbm.at[pl.ds(base, ROWS_PER_TILE)],
                    agg_sh.at[pl.ds(base, ROWS_PER_TILE)])
    plsc.subcore_barrier()

    def chunk_body(k, carry):
        cid = wid + k * NTILES

        @pl.when(cid < NCHUNKS)
        def _():
            off = cid * CHUNK
            pltpu.sync_copy(src_hbm.at[pl.ds(off, CHUNK)], src_v)
            pltpu.sync_copy(dst_hbm.at[pl.ds(off, CHUNK)], dst_v)
            pltpu.async_copy(h_hbm.at[src_v], rows_v, sem).wait()
            pltpu.sync_copy(e_hbm.at[pl.ds(off, CHUNK)], e_v)

            def row_body(r, carry2):
                for j in range(H // 16):
                    sl = pl.ds(j * 16, 16)
                    rows_v[r, sl] = jnp.maximum(rows_v[r, sl] + e_v[r, sl], 0.0)
                return carry2

            lax.fori_loop(0, CHUNK, row_body, 0, unroll=False)
            pltpu.sync_copy(rows_v, agg_sh.at[dst_v], add=True)

        return carry

    lax.fori_loop(0, CHUNKS_PER_TILE, chunk_body, 0, unroll=False)
    plsc.subcore_barrier()

    pltpu.sync_copy(agg_sh.at[pl.ds(base, ROWS_PER_TILE)],
                    out_hbm.at[c, pl.ds(base, ROWS_PER_TILE)])


def _sc_message_pass(h, e, src, dst, zeros):
    mesh = plsc.VectorSubcoreMesh(core_axis_name="c", subcore_axis_name="s")
    fn = pl.kernel(
        _sc_message_pass_body,
        out_type=jax.ShapeDtypeStruct((NC, N_PAD, H), jnp.float32),
        mesh=mesh,
        scratch_types=[
            pltpu.VMEM((CHUNK,), jnp.int32),
            pltpu.VMEM((CHUNK,), jnp.int32),
            pltpu.VMEM((CHUNK, H), jnp.float32),
            pltpu.VMEM((CHUNK, H), jnp.float32),
            pltpu.VMEM_SHARED((N_PAD, H), jnp.float32),
            pltpu.SemaphoreType.DMA,
        ],
    )
    return fn(h, e, src, dst, zeros)


def _edge_mlp_body(ea_ref, w_ref, b_ref, o0_ref, o1_ref, o2_ref):
    v = jnp.dot(ea_ref[...], w_ref[...],
                preferred_element_type=jnp.float32) + b_ref[...]
    o0_ref[...] = v[:, :H]
    o1_ref[...] = v[:, H:2 * H]
    o2_ref[...] = v[:, 2 * H:]


def _edge_mlp(edge_attr, w_cat, b_cat):
    BE = 4000
    grid = (E // BE,)
    out = jax.ShapeDtypeStruct((E, H), jnp.float32)
    return pl.pallas_call(
        _edge_mlp_body,
        grid=grid,
        in_specs=[
            pl.BlockSpec((BE, DE), lambda i: (i, 0)),
            pl.BlockSpec((DE, 3 * H), lambda i: (0, 0)),
            pl.BlockSpec((1, 3 * H), lambda i: (0, 0)),
        ],
        out_specs=[
            pl.BlockSpec((BE, H), lambda i: (i, 0)),
            pl.BlockSpec((BE, H), lambda i: (i, 0)),
            pl.BlockSpec((BE, H), lambda i: (i, 0)),
        ],
        out_shape=[out, out, out],
    )(edge_attr, w_cat, b_cat)


def _node_mlp_body(h_ref, part_ref, w1_ref, b1_ref, w2_ref, b2_ref,
                   lng_ref, lnb_ref, eps_ref, o_ref, *, residual):
    h = h_ref[...]
    agg = part_ref[0] + part_ref[1]
    z = (1.0 + eps_ref[0]) * h + agg
    z1 = jnp.dot(z, w1_ref[...], preferred_element_type=jnp.float32)
    z1 = jnp.maximum(z1 + b1_ref[...], 0.0)
    z2 = jnp.dot(z1, w2_ref[...], preferred_element_type=jnp.float32)
    z2 = z2 + b2_ref[...]
    mu = jnp.mean(z2, axis=-1, keepdims=True)
    var = jnp.mean((z2 - mu) ** 2, axis=-1, keepdims=True)
    zn = (z2 - mu) * lax.rsqrt(var + 1e-5) * lng_ref[...] + lnb_ref[...]
    zr = jnp.maximum(zn, 0.0)
    if residual:
        o_ref[...] = h + 0.3 * zr
    else:
        o_ref[...] = zr


def _node_mlp(h, part, w1, b1, w2, b2, lng, lnb, eps, residual):
    BN = 1000
    grid = (N // BN,)
    body = functools.partial(_node_mlp_body, residual=residual)
    return pl.pallas_call(
        body,
        grid=grid,
        in_specs=[
            pl.BlockSpec((BN, H), lambda i: (i, 0)),
            pl.BlockSpec((NC, BN, H), lambda i: (0, i, 0)),
            pl.BlockSpec((H, 2 * H), lambda i: (0, 0)),
            pl.BlockSpec((1, 2 * H), lambda i: (0, 0)),
            pl.BlockSpec((2 * H, H), lambda i: (0, 0)),
            pl.BlockSpec((1, H), lambda i: (0, 0)),
            pl.BlockSpec((1, H), lambda i: (0, 0)),
            pl.BlockSpec((1, H), lambda i: (0, 0)),
            pl.BlockSpec(memory_space=pltpu.SMEM),
        ],
        out_specs=pl.BlockSpec((BN, H), lambda i: (i, 0)),
        out_shape=jax.ShapeDtypeStruct((N, H), jnp.float32),
    )(h, part, w1, b1, w2, b2, lng, lnb, eps)


def kernel(x, edge_index, edge_attr,
           W_edge_0, b_edge_0, eps_0, W1_0, b1_0, bn1_g_0, bn1_b_0,
           W2_0, b2_0, bn_g_0, bn_b_0, ln_g_0, ln_b_0,
           W_edge_1, b_edge_1, eps_1, W1_1, b1_1, bn1_g_1, bn1_b_1,
           W2_1, b2_1, bn_g_1, bn_b_1, ln_g_1, ln_b_1,
           W_edge_2, b_edge_2, eps_2, W1_2, b1_2, bn1_g_2, bn1_b_2,
           W2_2, b2_2, bn_g_2, bn_b_2, ln_g_2, ln_b_2):
    bn_scale = 1.0 / jnp.sqrt(1.0 + 1e-5)
    src = edge_index[0]
    dst = edge_index[1]
    zeros = jnp.zeros((N_PAD, H), jnp.float32)

    Ws, Es = [], []
    for (W_e, b_e, eps, W1, b1, g1, bb1, W2, b2, g2, bb2, lg, lb) in (
        (W_edge_0, b_edge_0, eps_0, W1_0, b1_0, bn1_g_0, bn1_b_0, W2_0, b2_0,
         bn_g_0, bn_b_0, ln_g_0, ln_b_0),
        (W_edge_1, b_edge_1, eps_1, W1_1, b1_1, bn1_g_1, bn1_b_1, W2_1, b2_1,
         bn_g_1, bn_b_1, ln_g_1, ln_b_1),
        (W_edge_2, b_edge_2, eps_2, W1_2, b1_2, bn1_g_2, bn1_b_2, W2_2, b2_2,
         bn_g_2, bn_b_2, ln_g_2, ln_b_2),
    ):
        s1 = bn_scale * g1
        w1f = W1 * s1[None, :]
        b1f = (b1 * s1 + bb1)[None, :]
        s2 = bn_scale * g2
        w2f = W2 * s2[None, :]
        b2f = (b2 * s2 + bb2)[None, :]
        Ws.append((eps.reshape(1), w1f, b1f, w2f, b2f,
                   lg[None, :], lb[None, :]))
        Es.append((W_e, b_e))

    w_cat = jnp.concatenate([Es[0][0], Es[1][0], Es[2][0]], axis=1)
    b_cat = jnp.concatenate([Es[0][1], Es[1][1], Es[2][1]])[None, :]
    e0, e1, e2 = _edge_mlp(edge_attr, w_cat, b_cat)

    h = x
    for i, e in enumerate((e0, e1, e2)):
        eps, w1f, b1f, w2f, b2f, lg, lb = Ws[i]
        part = _sc_message_pass(h, e, src, dst, zeros)[:, :N]
        h = _node_mlp(h, part, w1f, b1f, w2f, b2f, lg, lb, eps,
                      residual=(i == 1))
    return h
